# baseline (speedup 1.0000x reference)
import jax
import jax.numpy as jnp
from jax.experimental import pallas as pl
from jax.experimental.pallas import tpu as pltpu

BATCH = 16384
NUM_CLASSES = 1000
ROW_CHUNK = 512
NUM_CHUNKS = BATCH // ROW_CHUNK
NUM_SLOTS = 8


def _onehot_multidma(x_ref, out_ref, vmem, sems):
    cols = jax.lax.broadcasted_iota(jnp.int32, (ROW_CHUNK, NUM_CLASSES), 1)

    def _copy(j, slot):
        return pltpu.make_async_copy(
            vmem.at[slot],
            out_ref.at[pl.ds(j * ROW_CHUNK, ROW_CHUNK), :],
            sems.at[slot],
        )

    for j in range(NUM_CHUNKS):
        slot = j % NUM_SLOTS
        if j >= NUM_SLOTS:
            _copy(j - NUM_SLOTS, slot).wait()
        idx = x_ref[pl.ds(j * ROW_CHUNK, ROW_CHUNK), :]
        vmem[slot, :, :] = (cols == idx).astype(jnp.float32)
        _copy(j, slot).start()

    for j in range(NUM_CHUNKS - NUM_SLOTS, NUM_CHUNKS):
        _copy(j, j % NUM_SLOTS).wait()


def kernel(x):
    x = x.astype(jnp.int32)
    return pl.pallas_call(
        _onehot_multidma,
        in_specs=[pl.BlockSpec(memory_space=pltpu.MemorySpace.VMEM)],
        out_specs=pl.BlockSpec(memory_space=pl.MemorySpace.ANY),
        out_shape=jax.ShapeDtypeStruct((BATCH, NUM_CLASSES), jnp.float32),
        scratch_shapes=[
            pltpu.VMEM((NUM_SLOTS, ROW_CHUNK, NUM_CLASSES), jnp.float32),
            pltpu.SemaphoreType.DMA((NUM_SLOTS,)),
        ],
    )(x)
